# baseline (speedup 1.0000x reference)
import functools

import jax
import jax.numpy as jnp
from jax import lax
from jax.experimental import pallas as pl
from jax.experimental.pallas import tpu as pltpu

_VMEM_LIMIT_BYTES = 48 * 1024 * 1024


def _const_spec(block_shape, index_map):
    try:
        return pl.BlockSpec(block_shape, index_map, pipeline_mode=pl.Buffered(1))
    except Exception:
        return pl.BlockSpec(block_shape, index_map)


def _frontend_kernel(len_ref, tok_ref, emb_ref, wc0_ref, wc1_ref, wc2_ref,
                     bc_ref, g_ref, be_ref, wi0_ref, wi1_ref, bi_ref, z_ref,
                     xpad_ref, *, depth, ksize, eps, neg_slope):
    Tp, Bb, C = xpad_ref.shape
    V = emb_ref.shape[0]
    p = (ksize - 1) // 2
    T = Tp - 2 * p
    BT = T * Bb

    lens = len_ref[...].reshape(1, Bb, 1)
    pos = lax.broadcasted_iota(jnp.int32, (T, Bb, 1), 0)
    keep = (pos < lens).reshape(BT, 1)

    tok = tok_ref[...].reshape(BT, 1)
    col = lax.broadcasted_iota(jnp.int32, (BT, V), 1)
    onehot = (col == tok).astype(jnp.bfloat16)
    x = jnp.dot(onehot, emb_ref[...],
                preferred_element_type=jnp.float32)
    x = jnp.where(keep, x, 0.0)

    if p > 0:
        xpad_ref[0:p] = jnp.zeros((p, Bb, C), jnp.bfloat16)
        xpad_ref[p + T:] = jnp.zeros((p, Bb, C), jnp.bfloat16)
    wc_refs = (wc0_ref, wc1_ref, wc2_ref)
    for d in range(depth):
        xpad_ref[p:p + T] = x.reshape(T, Bb, C).astype(jnp.bfloat16)
        xp = xpad_ref[...]
        xcol = jnp.concatenate(
            [xp[k:k + T].reshape(BT, C) for k in range(ksize)], axis=-1)
        acc = jnp.dot(xcol, wc_refs[d][...],
                      preferred_element_type=jnp.float32)
        acc = acc + bc_ref[d]
        mean = jnp.mean(acc, axis=-1, keepdims=True)
        var = jnp.mean(jnp.square(acc - mean), axis=-1, keepdims=True)
        y = (acc - mean) * lax.rsqrt(var + eps)
        y = y * g_ref[d] + be_ref[d]
        y = jnp.where(y >= 0.0, y, neg_slope * y)
        x = jnp.where(keep, y, 0.0)

    H8 = z_ref.shape[-1]
    H4 = H8 // 2
    xb = x.astype(jnp.bfloat16)
    zf = jnp.dot(xb, wi0_ref[...],
                 preferred_element_type=jnp.float32) + bi_ref[:, :H4]
    zb = jnp.dot(xb, wi1_ref[...],
                 preferred_element_type=jnp.float32) + bi_ref[:, H4:]
    z_ref[:, :, :H4] = zf.reshape(T, Bb, H4).astype(z_ref.dtype)
    z_ref[:, :, H4:] = zb.reshape(T, Bb, H4).astype(z_ref.dtype)


def _bilstm_kernel(len_ref, zf_ref, zb_ref, whf_ref, whb_ref, of_ref, ob_ref,
                   hf_ref, cf_ref, hb_ref, cb_ref, *, t_total):
    Tt, Bb, H4 = zf_ref.shape
    H = H4 // 4

    @pl.when(pl.program_id(0) == 0)
    def _():
        hf_ref[...] = jnp.zeros((Bb, H), jnp.bfloat16)
        hb_ref[...] = jnp.zeros((Bb, H), jnp.bfloat16)
        cf_ref[...] = jnp.zeros((Bb, H), jnp.float32)
        cb_ref[...] = jnp.zeros((Bb, H), jnp.float32)

    whf = whf_ref[...]
    whb = whb_ref[...]
    lens = len_ref[...].reshape(Bb, 1)
    t0 = pl.program_id(0) * Tt

    def gates(zt, r, c):
        i = jax.nn.sigmoid(zt[:, 0:H].astype(jnp.float32) + r[:, 0:H])
        f = jax.nn.sigmoid(zt[:, H:2 * H].astype(jnp.float32) + r[:, H:2 * H])
        g = jnp.tanh(zt[:, 2 * H:3 * H].astype(jnp.float32) + r[:, 2 * H:3 * H])
        o = jax.nn.sigmoid(zt[:, 3 * H:4 * H].astype(jnp.float32) + r[:, 3 * H:])
        c_new = f * c + i * g
        return o * jnp.tanh(c_new), c_new

    def substep(s, h_f, c_f, h_b, c_b, r_f, r_b):
        tf = t0 + s
        tb = (t_total - 1) - tf
        sb = Tt - 1 - s
        act_f = tf < lens
        act_b = tb < lens
        hf_new, cf_new = gates(zf_ref[pl.ds(s, 1)][0], r_f, c_f)
        of_ref[pl.ds(s, 1)] = jnp.where(act_f, hf_new,
                                        0.0).astype(of_ref.dtype)[None]
        h_f = jnp.where(act_f, hf_new.astype(jnp.bfloat16), h_f)
        c_f = jnp.where(act_f, cf_new, c_f)
        r_f = jnp.dot(h_f, whf, preferred_element_type=jnp.float32)
        hb_new, cb_new = gates(zb_ref[pl.ds(sb, 1)][0], r_b, c_b)
        ob_ref[pl.ds(sb, 1)] = jnp.where(act_b, hb_new,
                                         0.0).astype(ob_ref.dtype)[None]
        h_b = jnp.where(act_b, hb_new.astype(jnp.bfloat16), h_b)
        c_b = jnp.where(act_b, cb_new, c_b)
        r_b = jnp.dot(h_b, whb, preferred_element_type=jnp.float32)
        return h_f, c_f, h_b, c_b, r_f, r_b

    def step2(j, carry):
        s = 2 * j
        carry = substep(s, *carry)
        carry = substep(s + 1, *carry)
        return carry

    h_f = hf_ref[...]
    c_f = cf_ref[...]
    h_b = hb_ref[...]
    c_b = cb_ref[...]
    r_f = jnp.dot(h_f, whf, preferred_element_type=jnp.float32)
    r_b = jnp.dot(h_b, whb, preferred_element_type=jnp.float32)
    h_f, c_f, h_b, c_b, _, _ = lax.fori_loop(
        0, Tt // 2, step2, (h_f, c_f, h_b, c_b, r_f, r_b))
    hf_ref[...] = h_f
    cf_ref[...] = c_f
    hb_ref[...] = h_b
    cb_ref[...] = c_b


def kernel(embedding, cnn0_w_eff, cnn0_bias, cnn0_gamma, cnn0_beta,
           cnn1_w_eff, cnn1_bias, cnn1_gamma, cnn1_beta,
           cnn2_w_eff, cnn2_bias, cnn2_gamma, cnn2_beta,
           lstm_w_ih_f, lstm_w_hh_f, lstm_b_ih_f, lstm_b_hh_f,
           lstm_w_ih_b, lstm_w_hh_b, lstm_b_ih_b, lstm_b_hh_b,
           tokens, input_lengths, m):
    del m
    B, T = tokens.shape
    V, C = embedding.shape
    H = C // 2
    H4, H8 = 4 * H, 8 * H
    cnn = [(cnn0_w_eff, cnn0_bias, cnn0_gamma, cnn0_beta),
           (cnn1_w_eff, cnn1_bias, cnn1_gamma, cnn1_beta),
           (cnn2_w_eff, cnn2_bias, cnn2_gamma, cnn2_beta)]
    depth = len(cnn)
    ksize = cnn0_w_eff.shape[-1]
    p = (ksize - 1) // 2

    emb_bf16 = embedding.astype(jnp.bfloat16)
    wcs = [jnp.transpose(w, (2, 1, 0)).astype(jnp.bfloat16).reshape(ksize * C, C)
           for w, _, _, _ in cnn]
    bc = jnp.stack([b.reshape(1, C) for _, b, _, _ in cnn])
    g = jnp.stack([ga.reshape(1, C) for _, _, ga, _ in cnn])
    be = jnp.stack([bb.reshape(1, C) for _, _, _, bb in cnn])

    wi0 = lstm_w_ih_f.T.astype(jnp.bfloat16)
    wi1 = lstm_w_ih_b.T.astype(jnp.bfloat16)
    bi = jnp.concatenate([lstm_b_ih_f + lstm_b_hh_f,
                          lstm_b_ih_b + lstm_b_hh_b]).reshape(1, H8)
    whf = lstm_w_hh_f.T.astype(jnp.bfloat16)
    whb = lstm_w_hh_b.T.astype(jnp.bfloat16)

    tok3 = tokens.astype(jnp.int32).T.reshape(T, B, 1)
    lens3 = input_lengths.astype(jnp.int32).reshape(1, B, 1)

    Bblk = 16 if B % 16 == 0 else B
    nb = B // Bblk
    fe = functools.partial(_frontend_kernel, depth=depth, ksize=ksize,
                           eps=1e-5, neg_slope=0.2)
    z = pl.pallas_call(
        fe,
        grid=(nb,),
        in_specs=[
            pl.BlockSpec((1, Bblk, 1), lambda b: (0, b, 0)),
            pl.BlockSpec((T, Bblk, 1), lambda b: (0, b, 0)),
            _const_spec((V, C), lambda b: (0, 0)),
            _const_spec(wcs[0].shape, lambda b: (0, 0)),
            _const_spec(wcs[1].shape, lambda b: (0, 0)),
            _const_spec(wcs[2].shape, lambda b: (0, 0)),
            _const_spec(bc.shape, lambda b: (0, 0, 0)),
            _const_spec(g.shape, lambda b: (0, 0, 0)),
            _const_spec(be.shape, lambda b: (0, 0, 0)),
            _const_spec(wi0.shape, lambda b: (0, 0)),
            _const_spec(wi1.shape, lambda b: (0, 0)),
            _const_spec(bi.shape, lambda b: (0, 0)),
        ],
        out_specs=pl.BlockSpec((T, Bblk, H8), lambda b: (0, b, 0)),
        out_shape=jax.ShapeDtypeStruct((T, B, H8), jnp.bfloat16),
        scratch_shapes=[pltpu.VMEM((T + 2 * p, Bblk, C), jnp.bfloat16)],
        compiler_params=pltpu.CompilerParams(
            dimension_semantics=("arbitrary",),
            vmem_limit_bytes=_VMEM_LIMIT_BYTES),
    )(lens3, tok3, emb_bf16, wcs[0], wcs[1], wcs[2], bc, g, be, wi0, wi1, bi)

    nc = 4 if T % 8 == 0 else 1
    Tt = T // nc
    bl = functools.partial(_bilstm_kernel, t_total=T)
    of, ob = pl.pallas_call(
        bl,
        grid=(nc,),
        in_specs=[
            pl.BlockSpec((1, B, 1), lambda i: (0, 0, 0)),
            pl.BlockSpec((Tt, B, H4), lambda i: (i, 0, 0)),
            pl.BlockSpec((Tt, B, H4),
                         lambda i: (nc - 1 - i, 0, 1)),
            _const_spec((H, H4), lambda i: (0, 0)),
            _const_spec((H, H4), lambda i: (0, 0)),
        ],
        out_specs=[
            pl.BlockSpec((Tt, B, H), lambda i: (i, 0, 0)),
            pl.BlockSpec((Tt, B, H), lambda i: (nc - 1 - i, 0, 0)),
        ],
        out_shape=(jax.ShapeDtypeStruct((T, B, H), jnp.bfloat16),
                   jax.ShapeDtypeStruct((T, B, H), jnp.bfloat16)),
        scratch_shapes=[pltpu.VMEM((B, H), jnp.bfloat16),
                        pltpu.VMEM((B, H), jnp.float32),
                        pltpu.VMEM((B, H), jnp.bfloat16),
                        pltpu.VMEM((B, H), jnp.float32)],
        compiler_params=pltpu.CompilerParams(
            dimension_semantics=("arbitrary",),
            vmem_limit_bytes=_VMEM_LIMIT_BYTES),
    )(lens3, z, z, whf, whb)

    return jnp.concatenate(
        [jnp.transpose(of, (1, 2, 0)).astype(jnp.float32),
         jnp.transpose(ob, (1, 2, 0)).astype(jnp.float32)], axis=1)


# speedup vs baseline: 1.0004x; 1.0004x over previous
import functools

import jax
import jax.numpy as jnp
from jax import lax
from jax.experimental import pallas as pl
from jax.experimental.pallas import tpu as pltpu

_VMEM_LIMIT_BYTES = 48 * 1024 * 1024


def _const_spec(block_shape, index_map):
    try:
        return pl.BlockSpec(block_shape, index_map, pipeline_mode=pl.Buffered(1))
    except Exception:
        return pl.BlockSpec(block_shape, index_map)


def _frontend_kernel(len_ref, tok_ref, emb_ref, wc0_ref, wc1_ref, wc2_ref,
                     bc_ref, g_ref, be_ref, wi0_ref, wi1_ref, bi_ref,
                     zf_ref, zb_ref, xpad_ref, *, depth, ksize, eps, neg_slope):
    Tp, Bb, C = xpad_ref.shape
    V = emb_ref.shape[0]
    p = (ksize - 1) // 2
    T = Tp - 2 * p
    BT = T * Bb

    lens = len_ref[...].reshape(1, Bb, 1)
    pos = lax.broadcasted_iota(jnp.int32, (T, Bb, 1), 0)
    keep = (pos < lens).reshape(BT, 1)

    tok = tok_ref[...].reshape(BT, 1)
    col = lax.broadcasted_iota(jnp.int32, (BT, V), 1)
    onehot = (col == tok).astype(jnp.bfloat16)
    x = jnp.dot(onehot, emb_ref[...],
                preferred_element_type=jnp.float32)
    x = jnp.where(keep, x, 0.0)

    if p > 0:
        xpad_ref[0:p] = jnp.zeros((p, Bb, C), jnp.bfloat16)
        xpad_ref[p + T:] = jnp.zeros((p, Bb, C), jnp.bfloat16)
    wc_refs = (wc0_ref, wc1_ref, wc2_ref)
    for d in range(depth):
        xpad_ref[p:p + T] = x.reshape(T, Bb, C).astype(jnp.bfloat16)
        xp = xpad_ref[...]
        xcol = jnp.concatenate(
            [xp[k:k + T].reshape(BT, C) for k in range(ksize)], axis=-1)
        acc = jnp.dot(xcol, wc_refs[d][...],
                      preferred_element_type=jnp.float32)
        acc = acc + bc_ref[d]
        mean = jnp.mean(acc, axis=-1, keepdims=True)
        var = jnp.mean(jnp.square(acc - mean), axis=-1, keepdims=True)
        y = (acc - mean) * lax.rsqrt(var + eps)
        y = y * g_ref[d] + be_ref[d]
        y = jnp.where(y >= 0.0, y, neg_slope * y)
        x = jnp.where(keep, y, 0.0)

    H4 = zf_ref.shape[-1]
    xb = x.astype(jnp.bfloat16)
    zf = jnp.dot(xb, wi0_ref[...],
                 preferred_element_type=jnp.float32) + bi_ref[:, :H4]
    zb = jnp.dot(xb, wi1_ref[...],
                 preferred_element_type=jnp.float32) + bi_ref[:, H4:]
    zf_ref[...] = zf.reshape(T, Bb, H4).astype(zf_ref.dtype)
    zb_ref[...] = zb.reshape(T, Bb, H4).astype(zb_ref.dtype)


def _bilstm_kernel(len_ref, zf_ref, zb_ref, whf_ref, whb_ref, of_ref, ob_ref,
                   hf_ref, cf_ref, hb_ref, cb_ref, *, t_total):
    Tt, Bb, H4 = zf_ref.shape
    H = H4 // 4

    @pl.when(pl.program_id(0) == 0)
    def _():
        hf_ref[...] = jnp.zeros((Bb, H), jnp.bfloat16)
        hb_ref[...] = jnp.zeros((Bb, H), jnp.bfloat16)
        cf_ref[...] = jnp.zeros((Bb, H), jnp.float32)
        cb_ref[...] = jnp.zeros((Bb, H), jnp.float32)

    whf = whf_ref[...]
    whb = whb_ref[...]
    lens = len_ref[...].reshape(Bb, 1)
    t0 = pl.program_id(0) * Tt

    def gates(zt, r, c):
        i = jax.nn.sigmoid(zt[:, 0:H].astype(jnp.float32) + r[:, 0:H])
        f = jax.nn.sigmoid(zt[:, H:2 * H].astype(jnp.float32) + r[:, H:2 * H])
        g = jnp.tanh(zt[:, 2 * H:3 * H].astype(jnp.float32) + r[:, 2 * H:3 * H])
        o = jax.nn.sigmoid(zt[:, 3 * H:4 * H].astype(jnp.float32) + r[:, 3 * H:])
        c_new = f * c + i * g
        return o * jnp.tanh(c_new), c_new

    def substep(s, h_f, c_f, h_b, c_b, r_f, r_b):
        tf = t0 + s
        tb = (t_total - 1) - tf
        sb = Tt - 1 - s
        act_f = tf < lens
        act_b = tb < lens
        hf_new, cf_new = gates(zf_ref[pl.ds(s, 1)][0], r_f, c_f)
        of_ref[pl.ds(s, 1)] = jnp.where(act_f, hf_new,
                                        0.0).astype(of_ref.dtype)[None]
        h_f = jnp.where(act_f, hf_new.astype(jnp.bfloat16), h_f)
        c_f = jnp.where(act_f, cf_new, c_f)
        r_f = jnp.dot(h_f, whf, preferred_element_type=jnp.float32)
        hb_new, cb_new = gates(zb_ref[pl.ds(sb, 1)][0], r_b, c_b)
        ob_ref[pl.ds(sb, 1)] = jnp.where(act_b, hb_new,
                                         0.0).astype(ob_ref.dtype)[None]
        h_b = jnp.where(act_b, hb_new.astype(jnp.bfloat16), h_b)
        c_b = jnp.where(act_b, cb_new, c_b)
        r_b = jnp.dot(h_b, whb, preferred_element_type=jnp.float32)
        return h_f, c_f, h_b, c_b, r_f, r_b

    def step2(j, carry):
        s = 2 * j
        carry = substep(s, *carry)
        carry = substep(s + 1, *carry)
        return carry

    h_f = hf_ref[...]
    c_f = cf_ref[...]
    h_b = hb_ref[...]
    c_b = cb_ref[...]
    r_f = jnp.dot(h_f, whf, preferred_element_type=jnp.float32)
    r_b = jnp.dot(h_b, whb, preferred_element_type=jnp.float32)
    h_f, c_f, h_b, c_b, _, _ = lax.fori_loop(
        0, Tt // 2, step2, (h_f, c_f, h_b, c_b, r_f, r_b))
    hf_ref[...] = h_f
    cf_ref[...] = c_f
    hb_ref[...] = h_b
    cb_ref[...] = c_b


def kernel(embedding, cnn0_w_eff, cnn0_bias, cnn0_gamma, cnn0_beta,
           cnn1_w_eff, cnn1_bias, cnn1_gamma, cnn1_beta,
           cnn2_w_eff, cnn2_bias, cnn2_gamma, cnn2_beta,
           lstm_w_ih_f, lstm_w_hh_f, lstm_b_ih_f, lstm_b_hh_f,
           lstm_w_ih_b, lstm_w_hh_b, lstm_b_ih_b, lstm_b_hh_b,
           tokens, input_lengths, m):
    del m
    B, T = tokens.shape
    V, C = embedding.shape
    H = C // 2
    H4, H8 = 4 * H, 8 * H
    cnn = [(cnn0_w_eff, cnn0_bias, cnn0_gamma, cnn0_beta),
           (cnn1_w_eff, cnn1_bias, cnn1_gamma, cnn1_beta),
           (cnn2_w_eff, cnn2_bias, cnn2_gamma, cnn2_beta)]
    depth = len(cnn)
    ksize = cnn0_w_eff.shape[-1]
    p = (ksize - 1) // 2

    emb_bf16 = embedding.astype(jnp.bfloat16)
    wcs = [jnp.transpose(w, (2, 1, 0)).astype(jnp.bfloat16).reshape(ksize * C, C)
           for w, _, _, _ in cnn]
    bc = jnp.stack([b.reshape(1, C) for _, b, _, _ in cnn])
    g = jnp.stack([ga.reshape(1, C) for _, _, ga, _ in cnn])
    be = jnp.stack([bb.reshape(1, C) for _, _, _, bb in cnn])

    wi0 = lstm_w_ih_f.T.astype(jnp.bfloat16)
    wi1 = lstm_w_ih_b.T.astype(jnp.bfloat16)
    bi = jnp.concatenate([lstm_b_ih_f + lstm_b_hh_f,
                          lstm_b_ih_b + lstm_b_hh_b]).reshape(1, H8)
    whf = lstm_w_hh_f.T.astype(jnp.bfloat16)
    whb = lstm_w_hh_b.T.astype(jnp.bfloat16)

    tok3 = tokens.astype(jnp.int32).T.reshape(T, B, 1)
    lens3 = input_lengths.astype(jnp.int32).reshape(1, B, 1)

    Bblk = 16 if B % 16 == 0 else B
    nb = B // Bblk
    fe = functools.partial(_frontend_kernel, depth=depth, ksize=ksize,
                           eps=1e-5, neg_slope=0.2)
    zf_arr, zb_arr = pl.pallas_call(
        fe,
        grid=(nb,),
        in_specs=[
            pl.BlockSpec((1, Bblk, 1), lambda b: (0, b, 0)),
            pl.BlockSpec((T, Bblk, 1), lambda b: (0, b, 0)),
            _const_spec((V, C), lambda b: (0, 0)),
            _const_spec(wcs[0].shape, lambda b: (0, 0)),
            _const_spec(wcs[1].shape, lambda b: (0, 0)),
            _const_spec(wcs[2].shape, lambda b: (0, 0)),
            _const_spec(bc.shape, lambda b: (0, 0, 0)),
            _const_spec(g.shape, lambda b: (0, 0, 0)),
            _const_spec(be.shape, lambda b: (0, 0, 0)),
            _const_spec(wi0.shape, lambda b: (0, 0)),
            _const_spec(wi1.shape, lambda b: (0, 0)),
            _const_spec(bi.shape, lambda b: (0, 0)),
        ],
        out_specs=[pl.BlockSpec((T, Bblk, H4), lambda b: (0, b, 0)),
                   pl.BlockSpec((T, Bblk, H4), lambda b: (0, b, 0))],
        out_shape=(jax.ShapeDtypeStruct((T, B, H4), jnp.bfloat16),
                   jax.ShapeDtypeStruct((T, B, H4), jnp.bfloat16)),
        scratch_shapes=[pltpu.VMEM((T + 2 * p, Bblk, C), jnp.bfloat16)],
        compiler_params=pltpu.CompilerParams(
            dimension_semantics=("arbitrary",),
            vmem_limit_bytes=_VMEM_LIMIT_BYTES),
    )(lens3, tok3, emb_bf16, wcs[0], wcs[1], wcs[2], bc, g, be, wi0, wi1, bi)

    nc = 4 if T % 8 == 0 else 1
    Tt = T // nc
    bl = functools.partial(_bilstm_kernel, t_total=T)
    of, ob = pl.pallas_call(
        bl,
        grid=(nc,),
        in_specs=[
            pl.BlockSpec((1, B, 1), lambda i: (0, 0, 0)),
            pl.BlockSpec((Tt, B, H4), lambda i: (i, 0, 0)),
            pl.BlockSpec((Tt, B, H4),
                         lambda i: (nc - 1 - i, 0, 0)),
            _const_spec((H, H4), lambda i: (0, 0)),
            _const_spec((H, H4), lambda i: (0, 0)),
        ],
        out_specs=[
            pl.BlockSpec((Tt, B, H), lambda i: (i, 0, 0)),
            pl.BlockSpec((Tt, B, H), lambda i: (nc - 1 - i, 0, 0)),
        ],
        out_shape=(jax.ShapeDtypeStruct((T, B, H), jnp.bfloat16),
                   jax.ShapeDtypeStruct((T, B, H), jnp.bfloat16)),
        scratch_shapes=[pltpu.VMEM((B, H), jnp.bfloat16),
                        pltpu.VMEM((B, H), jnp.float32),
                        pltpu.VMEM((B, H), jnp.bfloat16),
                        pltpu.VMEM((B, H), jnp.float32)],
        compiler_params=pltpu.CompilerParams(
            dimension_semantics=("arbitrary",),
            vmem_limit_bytes=_VMEM_LIMIT_BYTES),
    )(lens3, zf_arr, zb_arr, whf, whb)

    return jnp.concatenate(
        [jnp.transpose(of, (1, 2, 0)).astype(jnp.float32),
         jnp.transpose(ob, (1, 2, 0)).astype(jnp.float32)], axis=1)


# speedup vs baseline: 1.0871x; 1.0867x over previous
import functools

import jax
import jax.numpy as jnp
from jax import lax
from jax.experimental import pallas as pl
from jax.experimental.pallas import tpu as pltpu

_VMEM_LIMIT_BYTES = 48 * 1024 * 1024


def _const_spec(block_shape, index_map):
    try:
        return pl.BlockSpec(block_shape, index_map, pipeline_mode=pl.Buffered(1))
    except Exception:
        return pl.BlockSpec(block_shape, index_map)


def _frontend_kernel(len_ref, tok_ref, emb_ref, wc_ref, bc_ref, g_ref, be_ref,
                     wi_ref, bi_ref, z_ref, xpad_ref, *, depth, ksize, eps,
                     neg_slope):
    Tp, Bb, C = xpad_ref.shape
    V = emb_ref.shape[0]
    p = (ksize - 1) // 2
    T = Tp - 2 * p
    BT = T * Bb

    lens = len_ref[...].reshape(1, Bb, 1)
    pos = lax.broadcasted_iota(jnp.int32, (T, Bb, 1), 0)
    keep = (pos < lens).reshape(BT, 1)

    tok = tok_ref[...].reshape(BT, 1)
    col = lax.broadcasted_iota(jnp.int32, (BT, V), 1)
    onehot = (col == tok).astype(jnp.bfloat16)
    x = jnp.dot(onehot, emb_ref[...],
                preferred_element_type=jnp.float32)
    x = jnp.where(keep, x, 0.0)

    if p > 0:
        xpad_ref[0:p] = jnp.zeros((p, Bb, C), jnp.bfloat16)
        xpad_ref[p + T:] = jnp.zeros((p, Bb, C), jnp.bfloat16)
    for d in range(depth):
        xpad_ref[p:p + T] = x.reshape(T, Bb, C).astype(jnp.bfloat16)
        xp = xpad_ref[...]
        xcol = jnp.concatenate(
            [xp[k:k + T].reshape(BT, C) for k in range(ksize)], axis=-1)
        acc = jnp.dot(xcol, wc_ref[d],
                      preferred_element_type=jnp.float32)
        acc = acc + bc_ref[d]
        mean = jnp.mean(acc, axis=-1, keepdims=True)
        var = jnp.mean(jnp.square(acc - mean), axis=-1, keepdims=True)
        y = (acc - mean) * lax.rsqrt(var + eps)
        y = y * g_ref[d] + be_ref[d]
        y = jnp.where(y >= 0.0, y, neg_slope * y)
        x = jnp.where(keep, y, 0.0)

    z = jnp.dot(x.astype(jnp.bfloat16), wi_ref[...],
                preferred_element_type=jnp.float32) + bi_ref[...]
    z_ref[...] = z.reshape(T, Bb, z_ref.shape[-1]).astype(z_ref.dtype)


def _bilstm_kernel(len_ref, zf_ref, zb_ref, wh_ref, of_ref, ob_ref,
                   hf_ref, cf_ref, hb_ref, cb_ref, *, t_total):
    Tt, Bb, H4 = zf_ref.shape
    H = H4 // 4

    @pl.when(pl.program_id(0) == 0)
    def _():
        hf_ref[...] = jnp.zeros((Bb, H), jnp.bfloat16)
        hb_ref[...] = jnp.zeros((Bb, H), jnp.bfloat16)
        cf_ref[...] = jnp.zeros((Bb, H), jnp.float32)
        cb_ref[...] = jnp.zeros((Bb, H), jnp.float32)

    whf = wh_ref[0]
    whb = wh_ref[1]
    lens = len_ref[...].reshape(Bb, 1)
    t0 = pl.program_id(0) * Tt

    def gates(zt, r, c):
        i = jax.nn.sigmoid(zt[:, 0:H].astype(jnp.float32) + r[:, 0:H])
        f = jax.nn.sigmoid(zt[:, H:2 * H].astype(jnp.float32) + r[:, H:2 * H])
        g = jnp.tanh(zt[:, 2 * H:3 * H].astype(jnp.float32) + r[:, 2 * H:3 * H])
        o = jax.nn.sigmoid(zt[:, 3 * H:4 * H].astype(jnp.float32) + r[:, 3 * H:])
        c_new = f * c + i * g
        return o * jnp.tanh(c_new), c_new

    def substep(s, h_f, c_f, h_b, c_b, r_f, r_b):
        tf = t0 + s
        tb = (t_total - 1) - tf
        sb = Tt - 1 - s
        act_f = tf < lens
        act_b = tb < lens
        hf_new, cf_new = gates(zf_ref[pl.ds(s, 1)][0], r_f, c_f)
        of_ref[pl.ds(s, 1)] = jnp.where(act_f, hf_new,
                                        0.0).astype(of_ref.dtype)[None]
        h_f = jnp.where(act_f, hf_new.astype(jnp.bfloat16), h_f)
        c_f = jnp.where(act_f, cf_new, c_f)
        r_f = jnp.dot(h_f, whf, preferred_element_type=jnp.float32)
        hb_new, cb_new = gates(zb_ref[pl.ds(sb, 1)][0], r_b, c_b)
        ob_ref[pl.ds(sb, 1)] = jnp.where(act_b, hb_new,
                                         0.0).astype(ob_ref.dtype)[None]
        h_b = jnp.where(act_b, hb_new.astype(jnp.bfloat16), h_b)
        c_b = jnp.where(act_b, cb_new, c_b)
        r_b = jnp.dot(h_b, whb, preferred_element_type=jnp.float32)
        return h_f, c_f, h_b, c_b, r_f, r_b

    def step2(j, carry):
        s = 2 * j
        carry = substep(s, *carry)
        carry = substep(s + 1, *carry)
        return carry

    h_f = hf_ref[...]
    c_f = cf_ref[...]
    h_b = hb_ref[...]
    c_b = cb_ref[...]
    r_f = jnp.dot(h_f, whf, preferred_element_type=jnp.float32)
    r_b = jnp.dot(h_b, whb, preferred_element_type=jnp.float32)
    h_f, c_f, h_b, c_b, _, _ = lax.fori_loop(
        0, Tt // 2, step2, (h_f, c_f, h_b, c_b, r_f, r_b))
    hf_ref[...] = h_f
    cf_ref[...] = c_f
    hb_ref[...] = h_b
    cb_ref[...] = c_b


def _epilogue_kernel(of_ref, ob_ref, out_ref):
    Bb, C2, T = out_ref.shape
    H = C2 // 2
    vf = of_ref[...]
    vb = ob_ref[...]
    for b in range(Bb):
        out_ref[b, 0:H, :] = jnp.transpose(vf[:, b, :],
                                           (1, 0)).astype(out_ref.dtype)
        out_ref[b, H:, :] = jnp.transpose(vb[:, b, :],
                                          (1, 0)).astype(out_ref.dtype)


def kernel(embedding, cnn0_w_eff, cnn0_bias, cnn0_gamma, cnn0_beta,
           cnn1_w_eff, cnn1_bias, cnn1_gamma, cnn1_beta,
           cnn2_w_eff, cnn2_bias, cnn2_gamma, cnn2_beta,
           lstm_w_ih_f, lstm_w_hh_f, lstm_b_ih_f, lstm_b_hh_f,
           lstm_w_ih_b, lstm_w_hh_b, lstm_b_ih_b, lstm_b_hh_b,
           tokens, input_lengths, m):
    del m
    B, T = tokens.shape
    V, C = embedding.shape
    H = C // 2
    H4, H8 = 4 * H, 8 * H
    cnn = [(cnn0_w_eff, cnn0_bias, cnn0_gamma, cnn0_beta),
           (cnn1_w_eff, cnn1_bias, cnn1_gamma, cnn1_beta),
           (cnn2_w_eff, cnn2_bias, cnn2_gamma, cnn2_beta)]
    depth = len(cnn)
    ksize = cnn0_w_eff.shape[-1]
    p = (ksize - 1) // 2

    emb_bf16 = embedding.astype(jnp.bfloat16)
    wc = jnp.transpose(jnp.stack([w for w, _, _, _ in cnn]),
                       (0, 3, 2, 1)).astype(jnp.bfloat16)
    wc = wc.reshape(depth, ksize * C, C)
    bc = jnp.stack([b.reshape(1, C) for _, b, _, _ in cnn])
    g = jnp.stack([ga.reshape(1, C) for _, _, ga, _ in cnn])
    be = jnp.stack([bb.reshape(1, C) for _, _, _, bb in cnn])

    wi = jnp.transpose(jnp.stack([lstm_w_ih_f, lstm_w_ih_b]),
                       (2, 0, 1)).reshape(C, H8).astype(jnp.bfloat16)
    bi = jnp.concatenate([lstm_b_ih_f + lstm_b_hh_f,
                          lstm_b_ih_b + lstm_b_hh_b]).reshape(1, H8)
    wh2 = jnp.transpose(jnp.stack([lstm_w_hh_f, lstm_w_hh_b]),
                        (0, 2, 1)).astype(jnp.bfloat16)

    tok3 = tokens.astype(jnp.int32).T.reshape(T, B, 1)
    lens3 = input_lengths.astype(jnp.int32).reshape(1, B, 1)

    Bblk = 16 if B % 16 == 0 else B
    nb = B // Bblk
    fe = functools.partial(_frontend_kernel, depth=depth, ksize=ksize,
                           eps=1e-5, neg_slope=0.2)
    z = pl.pallas_call(
        fe,
        grid=(nb,),
        in_specs=[
            pl.BlockSpec((1, Bblk, 1), lambda b: (0, b, 0)),
            pl.BlockSpec((T, Bblk, 1), lambda b: (0, b, 0)),
            _const_spec((V, C), lambda b: (0, 0)),
            _const_spec(wc.shape, lambda b: (0, 0, 0)),
            _const_spec(bc.shape, lambda b: (0, 0, 0)),
            _const_spec(g.shape, lambda b: (0, 0, 0)),
            _const_spec(be.shape, lambda b: (0, 0, 0)),
            _const_spec(wi.shape, lambda b: (0, 0)),
            _const_spec(bi.shape, lambda b: (0, 0)),
        ],
        out_specs=pl.BlockSpec((T, Bblk, H8), lambda b: (0, b, 0)),
        out_shape=jax.ShapeDtypeStruct((T, B, H8), jnp.bfloat16),
        scratch_shapes=[pltpu.VMEM((T + 2 * p, Bblk, C), jnp.bfloat16)],
        compiler_params=pltpu.CompilerParams(
            dimension_semantics=("arbitrary",),
            vmem_limit_bytes=_VMEM_LIMIT_BYTES),
    )(lens3, tok3, emb_bf16, wc, bc, g, be, wi, bi)

    nc = 4 if T % 8 == 0 else 1
    Tt = T // nc
    bl = functools.partial(_bilstm_kernel, t_total=T)
    of, ob = pl.pallas_call(
        bl,
        grid=(nc,),
        in_specs=[
            pl.BlockSpec((1, B, 1), lambda i: (0, 0, 0)),
            pl.BlockSpec((Tt, B, H4), lambda i: (i, 0, 0)),
            pl.BlockSpec((Tt, B, H4),
                         lambda i: (nc - 1 - i, 0, 1)),
            _const_spec((2, H, H4), lambda i: (0, 0, 0)),
        ],
        out_specs=[
            pl.BlockSpec((Tt, B, H), lambda i: (i, 0, 0)),
            pl.BlockSpec((Tt, B, H), lambda i: (nc - 1 - i, 0, 0)),
        ],
        out_shape=(jax.ShapeDtypeStruct((T, B, H), jnp.bfloat16),
                   jax.ShapeDtypeStruct((T, B, H), jnp.bfloat16)),
        scratch_shapes=[pltpu.VMEM((B, H), jnp.bfloat16),
                        pltpu.VMEM((B, H), jnp.float32),
                        pltpu.VMEM((B, H), jnp.bfloat16),
                        pltpu.VMEM((B, H), jnp.float32)],
        compiler_params=pltpu.CompilerParams(
            dimension_semantics=("arbitrary",),
            vmem_limit_bytes=_VMEM_LIMIT_BYTES),
    )(lens3, z, z, wh2)

    Bb3 = 16 if B % 16 == 0 else B
    nb3 = B // Bb3
    return pl.pallas_call(
        _epilogue_kernel,
        grid=(nb3,),
        in_specs=[
            pl.BlockSpec((T, Bb3, H), lambda b: (0, b, 0)),
            pl.BlockSpec((T, Bb3, H), lambda b: (0, b, 0)),
        ],
        out_specs=pl.BlockSpec((Bb3, 2 * H, T), lambda b: (b, 0, 0)),
        out_shape=jax.ShapeDtypeStruct((B, 2 * H, T), jnp.float32),
        compiler_params=pltpu.CompilerParams(
            dimension_semantics=("parallel",),
            vmem_limit_bytes=_VMEM_LIMIT_BYTES),
    )(of, ob)


# speedup vs baseline: 1.1240x; 1.0339x over previous
import functools

import jax
import jax.numpy as jnp
from jax import lax
from jax.experimental import pallas as pl
from jax.experimental.pallas import tpu as pltpu

_VMEM_LIMIT_BYTES = 48 * 1024 * 1024


def _const_spec(block_shape, index_map):
    try:
        return pl.BlockSpec(block_shape, index_map, pipeline_mode=pl.Buffered(1))
    except Exception:
        return pl.BlockSpec(block_shape, index_map)


def _frontend_kernel(len_ref, tok_ref, emb_ref, wc_ref, bc_ref, g_ref, be_ref,
                     wi_ref, bi_ref, z_ref, xpad_ref, *, depth, ksize, eps,
                     neg_slope):
    Tp, Bb, C = xpad_ref.shape
    V = emb_ref.shape[0]
    p = (ksize - 1) // 2
    T = Tp - 2 * p
    BT = T * Bb

    lens = len_ref[...].reshape(1, Bb, 1)
    pos = lax.broadcasted_iota(jnp.int32, (T, Bb, 1), 0)
    keep = (pos < lens).reshape(BT, 1)

    tok = tok_ref[...].reshape(BT, 1)
    col = lax.broadcasted_iota(jnp.int32, (BT, V), 1)
    onehot = (col == tok).astype(jnp.bfloat16)
    x = jnp.dot(onehot, emb_ref[...],
                preferred_element_type=jnp.float32)
    x = jnp.where(keep, x, 0.0)

    if p > 0:
        xpad_ref[0:p] = jnp.zeros((p, Bb, C), jnp.bfloat16)
        xpad_ref[p + T:] = jnp.zeros((p, Bb, C), jnp.bfloat16)
    for d in range(depth):
        xpad_ref[p:p + T] = x.reshape(T, Bb, C).astype(jnp.bfloat16)
        xp = xpad_ref[...]
        xcol = jnp.concatenate(
            [xp[k:k + T].reshape(BT, C) for k in range(ksize)], axis=-1)
        acc = jnp.dot(xcol, wc_ref[d],
                      preferred_element_type=jnp.float32)
        acc = acc + bc_ref[d]
        mean = jnp.mean(acc, axis=-1, keepdims=True)
        var = jnp.mean(jnp.square(acc - mean), axis=-1, keepdims=True)
        y = (acc - mean) * lax.rsqrt(var + eps)
        y = y * g_ref[d] + be_ref[d]
        y = jnp.where(y >= 0.0, y, neg_slope * y)
        x = jnp.where(keep, y, 0.0)

    z = jnp.dot(x.astype(jnp.bfloat16), wi_ref[...],
                preferred_element_type=jnp.float32) + bi_ref[...]
    z_ref[...] = z.reshape(T, Bb, z_ref.shape[-1]).astype(z_ref.dtype)


def _bilstm_kernel(len_ref, zf_ref, zb_ref, wh_ref, of_ref, ob_ref,
                   hf_ref, cf_ref, hb_ref, cb_ref, *, t_total):
    Tt, Bb, H4 = zf_ref.shape
    H = H4 // 4

    @pl.when(pl.program_id(0) == 0)
    def _():
        hf_ref[...] = jnp.zeros((Bb, H), jnp.bfloat16)
        hb_ref[...] = jnp.zeros((Bb, H), jnp.bfloat16)
        cf_ref[...] = jnp.zeros((Bb, H), jnp.float32)
        cb_ref[...] = jnp.zeros((Bb, H), jnp.float32)

    whf = wh_ref[0]
    whb = wh_ref[1]
    lens = len_ref[...].reshape(Bb, 1)
    t0 = pl.program_id(0) * Tt

    def gates(zt, r, c):
        i = jax.nn.sigmoid(zt[:, 0:H].astype(jnp.float32) + r[:, 0:H])
        f = jax.nn.sigmoid(zt[:, H:2 * H].astype(jnp.float32) + r[:, H:2 * H])
        g = jnp.tanh(zt[:, 2 * H:3 * H].astype(jnp.float32) + r[:, 2 * H:3 * H])
        o = jax.nn.sigmoid(zt[:, 3 * H:4 * H].astype(jnp.float32) + r[:, 3 * H:])
        c_new = f * c + i * g
        return o * jnp.tanh(c_new), c_new

    def substep(s, h_f, c_f, h_b, c_b, r_f, r_b):
        tf = t0 + s
        tb = (t_total - 1) - tf
        sb = Tt - 1 - s
        act_f = tf < lens
        act_b = tb < lens
        hf_new, cf_new = gates(zf_ref[pl.ds(s, 1)][0], r_f, c_f)
        of_ref[pl.ds(s, 1)] = jnp.where(act_f, hf_new,
                                        0.0).astype(of_ref.dtype)[None]
        h_f = jnp.where(act_f, hf_new.astype(jnp.bfloat16), h_f)
        c_f = jnp.where(act_f, cf_new, c_f)
        r_f = jnp.dot(h_f, whf, preferred_element_type=jnp.float32)
        hb_new, cb_new = gates(zb_ref[pl.ds(sb, 1)][0], r_b, c_b)
        ob_ref[pl.ds(sb, 1)] = jnp.where(act_b, hb_new,
                                         0.0).astype(ob_ref.dtype)[None]
        h_b = jnp.where(act_b, hb_new.astype(jnp.bfloat16), h_b)
        c_b = jnp.where(act_b, cb_new, c_b)
        r_b = jnp.dot(h_b, whb, preferred_element_type=jnp.float32)
        return h_f, c_f, h_b, c_b, r_f, r_b

    def step4(j, carry):
        s = 4 * j
        for u in range(4):
            carry = substep(s + u, *carry)
        return carry

    h_f = hf_ref[...]
    c_f = cf_ref[...]
    h_b = hb_ref[...]
    c_b = cb_ref[...]
    r_f = jnp.dot(h_f, whf, preferred_element_type=jnp.float32)
    r_b = jnp.dot(h_b, whb, preferred_element_type=jnp.float32)
    h_f, c_f, h_b, c_b, _, _ = lax.fori_loop(
        0, Tt // 4, step4, (h_f, c_f, h_b, c_b, r_f, r_b))
    hf_ref[...] = h_f
    cf_ref[...] = c_f
    hb_ref[...] = h_b
    cb_ref[...] = c_b


def _epilogue_kernel(of_ref, ob_ref, out_ref):
    Bb, C2, T = out_ref.shape
    H = C2 // 2
    vf = of_ref[...]
    vb = ob_ref[...]
    for b in range(Bb):
        out_ref[b, 0:H, :] = jnp.transpose(vf[:, b, :],
                                           (1, 0)).astype(out_ref.dtype)
        out_ref[b, H:, :] = jnp.transpose(vb[:, b, :],
                                          (1, 0)).astype(out_ref.dtype)


def kernel(embedding, cnn0_w_eff, cnn0_bias, cnn0_gamma, cnn0_beta,
           cnn1_w_eff, cnn1_bias, cnn1_gamma, cnn1_beta,
           cnn2_w_eff, cnn2_bias, cnn2_gamma, cnn2_beta,
           lstm_w_ih_f, lstm_w_hh_f, lstm_b_ih_f, lstm_b_hh_f,
           lstm_w_ih_b, lstm_w_hh_b, lstm_b_ih_b, lstm_b_hh_b,
           tokens, input_lengths, m):
    del m
    B, T = tokens.shape
    V, C = embedding.shape
    H = C // 2
    H4, H8 = 4 * H, 8 * H
    cnn = [(cnn0_w_eff, cnn0_bias, cnn0_gamma, cnn0_beta),
           (cnn1_w_eff, cnn1_bias, cnn1_gamma, cnn1_beta),
           (cnn2_w_eff, cnn2_bias, cnn2_gamma, cnn2_beta)]
    depth = len(cnn)
    ksize = cnn0_w_eff.shape[-1]
    p = (ksize - 1) // 2

    emb_bf16 = embedding.astype(jnp.bfloat16)
    wc = jnp.transpose(jnp.stack([w for w, _, _, _ in cnn]),
                       (0, 3, 2, 1)).astype(jnp.bfloat16)
    wc = wc.reshape(depth, ksize * C, C)
    bc = jnp.stack([b.reshape(1, C) for _, b, _, _ in cnn])
    g = jnp.stack([ga.reshape(1, C) for _, _, ga, _ in cnn])
    be = jnp.stack([bb.reshape(1, C) for _, _, _, bb in cnn])

    wi = jnp.transpose(jnp.stack([lstm_w_ih_f, lstm_w_ih_b]),
                       (2, 0, 1)).reshape(C, H8).astype(jnp.bfloat16)
    bi = jnp.concatenate([lstm_b_ih_f + lstm_b_hh_f,
                          lstm_b_ih_b + lstm_b_hh_b]).reshape(1, H8)
    wh2 = jnp.transpose(jnp.stack([lstm_w_hh_f, lstm_w_hh_b]),
                        (0, 2, 1)).astype(jnp.bfloat16)

    tok3 = tokens.astype(jnp.int32).T.reshape(T, B, 1)
    lens3 = input_lengths.astype(jnp.int32).reshape(1, B, 1)

    Bblk = 16 if B % 16 == 0 else B
    nb = B // Bblk
    fe = functools.partial(_frontend_kernel, depth=depth, ksize=ksize,
                           eps=1e-5, neg_slope=0.2)
    z = pl.pallas_call(
        fe,
        grid=(nb,),
        in_specs=[
            pl.BlockSpec((1, Bblk, 1), lambda b: (0, b, 0)),
            pl.BlockSpec((T, Bblk, 1), lambda b: (0, b, 0)),
            _const_spec((V, C), lambda b: (0, 0)),
            _const_spec(wc.shape, lambda b: (0, 0, 0)),
            _const_spec(bc.shape, lambda b: (0, 0, 0)),
            _const_spec(g.shape, lambda b: (0, 0, 0)),
            _const_spec(be.shape, lambda b: (0, 0, 0)),
            _const_spec(wi.shape, lambda b: (0, 0)),
            _const_spec(bi.shape, lambda b: (0, 0)),
        ],
        out_specs=pl.BlockSpec((T, Bblk, H8), lambda b: (0, b, 0)),
        out_shape=jax.ShapeDtypeStruct((T, B, H8), jnp.bfloat16),
        scratch_shapes=[pltpu.VMEM((T + 2 * p, Bblk, C), jnp.bfloat16)],
        compiler_params=pltpu.CompilerParams(
            dimension_semantics=("arbitrary",),
            vmem_limit_bytes=_VMEM_LIMIT_BYTES),
    )(lens3, tok3, emb_bf16, wc, bc, g, be, wi, bi)

    nc = 4 if T % 8 == 0 else 1
    Tt = T // nc
    bl = functools.partial(_bilstm_kernel, t_total=T)
    of, ob = pl.pallas_call(
        bl,
        grid=(nc,),
        in_specs=[
            pl.BlockSpec((1, B, 1), lambda i: (0, 0, 0)),
            pl.BlockSpec((Tt, B, H4), lambda i: (i, 0, 0)),
            pl.BlockSpec((Tt, B, H4),
                         lambda i: (nc - 1 - i, 0, 1)),
            _const_spec((2, H, H4), lambda i: (0, 0, 0)),
        ],
        out_specs=[
            pl.BlockSpec((Tt, B, H), lambda i: (i, 0, 0)),
            pl.BlockSpec((Tt, B, H), lambda i: (nc - 1 - i, 0, 0)),
        ],
        out_shape=(jax.ShapeDtypeStruct((T, B, H), jnp.bfloat16),
                   jax.ShapeDtypeStruct((T, B, H), jnp.bfloat16)),
        scratch_shapes=[pltpu.VMEM((B, H), jnp.bfloat16),
                        pltpu.VMEM((B, H), jnp.float32),
                        pltpu.VMEM((B, H), jnp.bfloat16),
                        pltpu.VMEM((B, H), jnp.float32)],
        compiler_params=pltpu.CompilerParams(
            dimension_semantics=("arbitrary",),
            vmem_limit_bytes=_VMEM_LIMIT_BYTES),
    )(lens3, z, z, wh2)

    Bb3 = 16 if B % 16 == 0 else B
    nb3 = B // Bb3
    return pl.pallas_call(
        _epilogue_kernel,
        grid=(nb3,),
        in_specs=[
            pl.BlockSpec((T, Bb3, H), lambda b: (0, b, 0)),
            pl.BlockSpec((T, Bb3, H), lambda b: (0, b, 0)),
        ],
        out_specs=pl.BlockSpec((Bb3, 2 * H, T), lambda b: (b, 0, 0)),
        out_shape=jax.ShapeDtypeStruct((B, 2 * H, T), jnp.float32),
        compiler_params=pltpu.CompilerParams(
            dimension_semantics=("parallel",),
            vmem_limit_bytes=_VMEM_LIMIT_BYTES),
    )(of, ob)


# speedup vs baseline: 1.1407x; 1.0149x over previous
import functools

import jax
import jax.numpy as jnp
from jax import lax
from jax.experimental import pallas as pl
from jax.experimental.pallas import tpu as pltpu

_VMEM_LIMIT_BYTES = 48 * 1024 * 1024


def _const_spec(block_shape, index_map):
    try:
        return pl.BlockSpec(block_shape, index_map, pipeline_mode=pl.Buffered(1))
    except Exception:
        return pl.BlockSpec(block_shape, index_map)


def _frontend_kernel(len_ref, tok_ref, emb_ref, wc_ref, bc_ref, g_ref, be_ref,
                     wi_ref, bi_ref, z_ref, xpad_ref, *, depth, ksize, eps,
                     neg_slope):
    Tp, Bb, C = xpad_ref.shape
    V = emb_ref.shape[0]
    p = (ksize - 1) // 2
    T = Tp - 2 * p
    BT = T * Bb

    lens = len_ref[...].reshape(1, Bb, 1)
    pos = lax.broadcasted_iota(jnp.int32, (T, Bb, 1), 0)
    keep = (pos < lens).reshape(BT, 1)

    tok = tok_ref[...].reshape(BT, 1)
    col = lax.broadcasted_iota(jnp.int32, (BT, V), 1)
    onehot = (col == tok).astype(jnp.bfloat16)
    x = jnp.dot(onehot, emb_ref[...],
                preferred_element_type=jnp.float32)
    x = jnp.where(keep, x, 0.0)

    if p > 0:
        xpad_ref[0:p] = jnp.zeros((p, Bb, C), jnp.bfloat16)
        xpad_ref[p + T:] = jnp.zeros((p, Bb, C), jnp.bfloat16)
    for d in range(depth):
        xpad_ref[p:p + T] = x.reshape(T, Bb, C).astype(jnp.bfloat16)
        xp = xpad_ref[...]
        xcol = jnp.concatenate(
            [xp[k:k + T].reshape(BT, C) for k in range(ksize)], axis=-1)
        acc = jnp.dot(xcol, wc_ref[d],
                      preferred_element_type=jnp.float32)
        acc = acc + bc_ref[d]
        mean = jnp.mean(acc, axis=-1, keepdims=True)
        var = jnp.mean(jnp.square(acc - mean), axis=-1, keepdims=True)
        y = (acc - mean) * lax.rsqrt(var + eps)
        y = y * g_ref[d] + be_ref[d]
        y = jnp.where(y >= 0.0, y, neg_slope * y)
        x = jnp.where(keep, y, 0.0)

    z = jnp.dot(x.astype(jnp.bfloat16), wi_ref[...],
                preferred_element_type=jnp.float32) + bi_ref[...]
    z_ref[...] = z.reshape(T, Bb, z_ref.shape[-1]).astype(z_ref.dtype)


def _bilstm_kernel(len_ref, zf_ref, zb_ref, wh_ref, of_ref, ob_ref,
                   hf_ref, cf_ref, hb_ref, cb_ref, *, t_total):
    Tt, Bb, H4 = zf_ref.shape
    H = H4 // 4

    @pl.when(pl.program_id(0) == 0)
    def _():
        hf_ref[...] = jnp.zeros((Bb, H), jnp.bfloat16)
        hb_ref[...] = jnp.zeros((Bb, H), jnp.bfloat16)
        cf_ref[...] = jnp.zeros((Bb, H), jnp.float32)
        cb_ref[...] = jnp.zeros((Bb, H), jnp.float32)

    whf = wh_ref[0]
    whb = wh_ref[1]
    lens = len_ref[...].reshape(Bb, 1)
    t0 = pl.program_id(0) * Tt

    def gates(zt, r, c):
        i = jax.nn.sigmoid(zt[:, 0:H].astype(jnp.float32) + r[:, 0:H])
        f = jax.nn.sigmoid(zt[:, H:2 * H].astype(jnp.float32) + r[:, H:2 * H])
        g = jnp.tanh(zt[:, 2 * H:3 * H].astype(jnp.float32) + r[:, 2 * H:3 * H])
        o = jax.nn.sigmoid(zt[:, 3 * H:4 * H].astype(jnp.float32) + r[:, 3 * H:])
        c_new = f * c + i * g
        return o * jnp.tanh(c_new), c_new

    def substep(s, h_f, c_f, h_b, c_b, r_f, r_b):
        tf = t0 + s
        tb = (t_total - 1) - tf
        sb = Tt - 1 - s
        act_f = tf < lens
        act_b = tb < lens
        hf_new, cf_new = gates(zf_ref[pl.ds(s, 1)][0], r_f, c_f)
        of_ref[pl.ds(s, 1)] = jnp.where(act_f, hf_new,
                                        0.0).astype(of_ref.dtype)[None]
        h_f = jnp.where(act_f, hf_new.astype(jnp.bfloat16), h_f)
        c_f = jnp.where(act_f, cf_new, c_f)
        r_f = jnp.dot(h_f, whf, preferred_element_type=jnp.float32)
        hb_new, cb_new = gates(zb_ref[pl.ds(sb, 1)][0], r_b, c_b)
        ob_ref[pl.ds(sb, 1)] = jnp.where(act_b, hb_new,
                                         0.0).astype(ob_ref.dtype)[None]
        h_b = jnp.where(act_b, hb_new.astype(jnp.bfloat16), h_b)
        c_b = jnp.where(act_b, cb_new, c_b)
        r_b = jnp.dot(h_b, whb, preferred_element_type=jnp.float32)
        return h_f, c_f, h_b, c_b, r_f, r_b

    def step8(j, carry):
        s = 8 * j
        for u in range(8):
            carry = substep(s + u, *carry)
        return carry

    h_f = hf_ref[...]
    c_f = cf_ref[...]
    h_b = hb_ref[...]
    c_b = cb_ref[...]
    r_f = jnp.dot(h_f, whf, preferred_element_type=jnp.float32)
    r_b = jnp.dot(h_b, whb, preferred_element_type=jnp.float32)
    h_f, c_f, h_b, c_b, _, _ = lax.fori_loop(
        0, Tt // 8, step8, (h_f, c_f, h_b, c_b, r_f, r_b))
    hf_ref[...] = h_f
    cf_ref[...] = c_f
    hb_ref[...] = h_b
    cb_ref[...] = c_b


def _epilogue_kernel(of_ref, ob_ref, out_ref):
    Bb, C2, T = out_ref.shape
    H = C2 // 2
    vf = of_ref[...]
    vb = ob_ref[...]
    for b in range(Bb):
        out_ref[b, 0:H, :] = jnp.transpose(vf[:, b, :],
                                           (1, 0)).astype(out_ref.dtype)
        out_ref[b, H:, :] = jnp.transpose(vb[:, b, :],
                                          (1, 0)).astype(out_ref.dtype)


def kernel(embedding, cnn0_w_eff, cnn0_bias, cnn0_gamma, cnn0_beta,
           cnn1_w_eff, cnn1_bias, cnn1_gamma, cnn1_beta,
           cnn2_w_eff, cnn2_bias, cnn2_gamma, cnn2_beta,
           lstm_w_ih_f, lstm_w_hh_f, lstm_b_ih_f, lstm_b_hh_f,
           lstm_w_ih_b, lstm_w_hh_b, lstm_b_ih_b, lstm_b_hh_b,
           tokens, input_lengths, m):
    del m
    B, T = tokens.shape
    V, C = embedding.shape
    H = C // 2
    H4, H8 = 4 * H, 8 * H
    cnn = [(cnn0_w_eff, cnn0_bias, cnn0_gamma, cnn0_beta),
           (cnn1_w_eff, cnn1_bias, cnn1_gamma, cnn1_beta),
           (cnn2_w_eff, cnn2_bias, cnn2_gamma, cnn2_beta)]
    depth = len(cnn)
    ksize = cnn0_w_eff.shape[-1]
    p = (ksize - 1) // 2

    emb_bf16 = embedding.astype(jnp.bfloat16)
    wc = jnp.transpose(jnp.stack([w for w, _, _, _ in cnn]),
                       (0, 3, 2, 1)).astype(jnp.bfloat16)
    wc = wc.reshape(depth, ksize * C, C)
    bc = jnp.stack([b.reshape(1, C) for _, b, _, _ in cnn])
    g = jnp.stack([ga.reshape(1, C) for _, _, ga, _ in cnn])
    be = jnp.stack([bb.reshape(1, C) for _, _, _, bb in cnn])

    wi = jnp.transpose(jnp.stack([lstm_w_ih_f, lstm_w_ih_b]),
                       (2, 0, 1)).reshape(C, H8).astype(jnp.bfloat16)
    bi = jnp.concatenate([lstm_b_ih_f + lstm_b_hh_f,
                          lstm_b_ih_b + lstm_b_hh_b]).reshape(1, H8)
    wh2 = jnp.transpose(jnp.stack([lstm_w_hh_f, lstm_w_hh_b]),
                        (0, 2, 1)).astype(jnp.bfloat16)

    tok3 = tokens.astype(jnp.int32).T.reshape(T, B, 1)
    lens3 = input_lengths.astype(jnp.int32).reshape(1, B, 1)

    Bblk = 16 if B % 16 == 0 else B
    nb = B // Bblk
    fe = functools.partial(_frontend_kernel, depth=depth, ksize=ksize,
                           eps=1e-5, neg_slope=0.2)
    z = pl.pallas_call(
        fe,
        grid=(nb,),
        in_specs=[
            pl.BlockSpec((1, Bblk, 1), lambda b: (0, b, 0)),
            pl.BlockSpec((T, Bblk, 1), lambda b: (0, b, 0)),
            _const_spec((V, C), lambda b: (0, 0)),
            _const_spec(wc.shape, lambda b: (0, 0, 0)),
            _const_spec(bc.shape, lambda b: (0, 0, 0)),
            _const_spec(g.shape, lambda b: (0, 0, 0)),
            _const_spec(be.shape, lambda b: (0, 0, 0)),
            _const_spec(wi.shape, lambda b: (0, 0)),
            _const_spec(bi.shape, lambda b: (0, 0)),
        ],
        out_specs=pl.BlockSpec((T, Bblk, H8), lambda b: (0, b, 0)),
        out_shape=jax.ShapeDtypeStruct((T, B, H8), jnp.bfloat16),
        scratch_shapes=[pltpu.VMEM((T + 2 * p, Bblk, C), jnp.bfloat16)],
        compiler_params=pltpu.CompilerParams(
            dimension_semantics=("arbitrary",),
            vmem_limit_bytes=_VMEM_LIMIT_BYTES),
    )(lens3, tok3, emb_bf16, wc, bc, g, be, wi, bi)

    nc = 4 if T % 8 == 0 else 1
    Tt = T // nc
    bl = functools.partial(_bilstm_kernel, t_total=T)
    of, ob = pl.pallas_call(
        bl,
        grid=(nc,),
        in_specs=[
            pl.BlockSpec((1, B, 1), lambda i: (0, 0, 0)),
            pl.BlockSpec((Tt, B, H4), lambda i: (i, 0, 0)),
            pl.BlockSpec((Tt, B, H4),
                         lambda i: (nc - 1 - i, 0, 1)),
            _const_spec((2, H, H4), lambda i: (0, 0, 0)),
        ],
        out_specs=[
            pl.BlockSpec((Tt, B, H), lambda i: (i, 0, 0)),
            pl.BlockSpec((Tt, B, H), lambda i: (nc - 1 - i, 0, 0)),
        ],
        out_shape=(jax.ShapeDtypeStruct((T, B, H), jnp.bfloat16),
                   jax.ShapeDtypeStruct((T, B, H), jnp.bfloat16)),
        scratch_shapes=[pltpu.VMEM((B, H), jnp.bfloat16),
                        pltpu.VMEM((B, H), jnp.float32),
                        pltpu.VMEM((B, H), jnp.bfloat16),
                        pltpu.VMEM((B, H), jnp.float32)],
        compiler_params=pltpu.CompilerParams(
            dimension_semantics=("arbitrary",),
            vmem_limit_bytes=_VMEM_LIMIT_BYTES),
    )(lens3, z, z, wh2)

    Bb3 = 16 if B % 16 == 0 else B
    nb3 = B // Bb3
    return pl.pallas_call(
        _epilogue_kernel,
        grid=(nb3,),
        in_specs=[
            pl.BlockSpec((T, Bb3, H), lambda b: (0, b, 0)),
            pl.BlockSpec((T, Bb3, H), lambda b: (0, b, 0)),
        ],
        out_specs=pl.BlockSpec((Bb3, 2 * H, T), lambda b: (b, 0, 0)),
        out_shape=jax.ShapeDtypeStruct((B, 2 * H, T), jnp.float32),
        compiler_params=pltpu.CompilerParams(
            dimension_semantics=("parallel",),
            vmem_limit_bytes=_VMEM_LIMIT_BYTES),
    )(of, ob)


# speedup vs baseline: 1.1786x; 1.0332x over previous
import functools

import jax
import jax.numpy as jnp
from jax import lax
from jax.experimental import pallas as pl
from jax.experimental.pallas import tpu as pltpu

_VMEM_LIMIT_BYTES = 48 * 1024 * 1024


def _const_spec(block_shape, index_map):
    try:
        return pl.BlockSpec(block_shape, index_map, pipeline_mode=pl.Buffered(1))
    except Exception:
        return pl.BlockSpec(block_shape, index_map)


def _frontend_kernel(len_ref, tok_ref, emb_ref, wc_ref, bc_ref, g_ref, be_ref,
                     wi_ref, bi_ref, z_ref, xpad_ref, *, depth, ksize, eps,
                     neg_slope):
    Tp, Bb, C = xpad_ref.shape
    V = emb_ref.shape[0]
    p = (ksize - 1) // 2
    T = Tp - 2 * p
    BT = T * Bb

    lens = len_ref[...].reshape(1, Bb, 1)
    pos = lax.broadcasted_iota(jnp.int32, (T, Bb, 1), 0)
    keep = (pos < lens).reshape(BT, 1)

    tok = tok_ref[...].reshape(BT, 1)
    col = lax.broadcasted_iota(jnp.int32, (BT, V), 1)
    onehot = (col == tok).astype(jnp.bfloat16)
    x = jnp.dot(onehot, emb_ref[...],
                preferred_element_type=jnp.float32)
    x = jnp.where(keep, x, 0.0)

    if p > 0:
        xpad_ref[0:p] = jnp.zeros((p, Bb, C), jnp.bfloat16)
        xpad_ref[p + T:] = jnp.zeros((p, Bb, C), jnp.bfloat16)
    for d in range(depth):
        xpad_ref[p:p + T] = x.reshape(T, Bb, C).astype(jnp.bfloat16)
        xp = xpad_ref[...]
        xcol = jnp.concatenate(
            [xp[k:k + T].reshape(BT, C) for k in range(ksize)], axis=-1)
        acc = jnp.dot(xcol, wc_ref[d],
                      preferred_element_type=jnp.float32)
        acc = acc + bc_ref[d]
        mean = jnp.mean(acc, axis=-1, keepdims=True)
        var = jnp.mean(jnp.square(acc - mean), axis=-1, keepdims=True)
        y = (acc - mean) * lax.rsqrt(var + eps)
        y = y * g_ref[d] + be_ref[d]
        y = jnp.where(y >= 0.0, y, neg_slope * y)
        x = jnp.where(keep, y, 0.0)

    z = jnp.dot(x.astype(jnp.bfloat16), wi_ref[...],
                preferred_element_type=jnp.float32) + bi_ref[...]
    z_ref[...] = z.reshape(T, Bb, z_ref.shape[-1]).astype(z_ref.dtype)


def _bilstm_kernel(len_ref, zf_ref, zb_ref, wh_ref, of_ref, ob_ref,
                   hf_ref, cf_ref, hb_ref, cb_ref, *, t_total):
    Tt, Bb, H4 = zf_ref.shape
    H = H4 // 4

    @pl.when(pl.program_id(0) == 0)
    def _():
        hf_ref[...] = jnp.zeros((Bb, H), jnp.bfloat16)
        hb_ref[...] = jnp.zeros((Bb, H), jnp.bfloat16)
        cf_ref[...] = jnp.zeros((Bb, H), jnp.float32)
        cb_ref[...] = jnp.zeros((Bb, H), jnp.float32)

    whf = wh_ref[0]
    whb = wh_ref[1]
    lens = len_ref[...].reshape(Bb, 1)
    t0 = pl.program_id(0) * Tt

    def gates(zt, r, c):
        i = jax.nn.sigmoid(zt[:, 0:H].astype(jnp.float32) + r[:, 0:H])
        f = jax.nn.sigmoid(zt[:, H:2 * H].astype(jnp.float32) + r[:, H:2 * H])
        g = jnp.tanh(zt[:, 2 * H:3 * H].astype(jnp.float32) + r[:, 2 * H:3 * H])
        o = jax.nn.sigmoid(zt[:, 3 * H:4 * H].astype(jnp.float32) + r[:, 3 * H:])
        c_new = f * c + i * g
        return o * jnp.tanh(c_new), c_new

    def substep(s, h_f, c_f, h_b, c_b, r_f, r_b):
        tf = t0 + s
        tb = (t_total - 1) - tf
        sb = Tt - 1 - s
        act_f = tf < lens
        act_b = tb < lens
        hf_new, cf_new = gates(zf_ref[pl.ds(s, 1)][0], r_f, c_f)
        of_ref[pl.ds(s, 1)] = jnp.where(act_f, hf_new,
                                        0.0).astype(of_ref.dtype)[None]
        h_f = jnp.where(act_f, hf_new.astype(jnp.bfloat16), h_f)
        c_f = jnp.where(act_f, cf_new, c_f)
        r_f = jnp.dot(h_f, whf, preferred_element_type=jnp.float32)
        hb_new, cb_new = gates(zb_ref[pl.ds(sb, 1)][0], r_b, c_b)
        ob_ref[pl.ds(sb, 1)] = jnp.where(act_b, hb_new,
                                         0.0).astype(ob_ref.dtype)[None]
        h_b = jnp.where(act_b, hb_new.astype(jnp.bfloat16), h_b)
        c_b = jnp.where(act_b, cb_new, c_b)
        r_b = jnp.dot(h_b, whb, preferred_element_type=jnp.float32)
        return h_f, c_f, h_b, c_b, r_f, r_b

    h_f = hf_ref[...]
    c_f = cf_ref[...]
    h_b = hb_ref[...]
    c_b = cb_ref[...]
    r_f = jnp.dot(h_f, whf, preferred_element_type=jnp.float32)
    r_b = jnp.dot(h_b, whb, preferred_element_type=jnp.float32)
    carry = (h_f, c_f, h_b, c_b, r_f, r_b)
    for s in range(Tt):
        carry = substep(s, *carry)
    h_f, c_f, h_b, c_b, _, _ = carry
    hf_ref[...] = h_f
    cf_ref[...] = c_f
    hb_ref[...] = h_b
    cb_ref[...] = c_b


def _epilogue_kernel(of_ref, ob_ref, out_ref):
    Bb, C2, T = out_ref.shape
    H = C2 // 2
    vf = of_ref[...]
    vb = ob_ref[...]
    for b in range(Bb):
        out_ref[b, 0:H, :] = jnp.transpose(vf[:, b, :],
                                           (1, 0)).astype(out_ref.dtype)
        out_ref[b, H:, :] = jnp.transpose(vb[:, b, :],
                                          (1, 0)).astype(out_ref.dtype)


def kernel(embedding, cnn0_w_eff, cnn0_bias, cnn0_gamma, cnn0_beta,
           cnn1_w_eff, cnn1_bias, cnn1_gamma, cnn1_beta,
           cnn2_w_eff, cnn2_bias, cnn2_gamma, cnn2_beta,
           lstm_w_ih_f, lstm_w_hh_f, lstm_b_ih_f, lstm_b_hh_f,
           lstm_w_ih_b, lstm_w_hh_b, lstm_b_ih_b, lstm_b_hh_b,
           tokens, input_lengths, m):
    del m
    B, T = tokens.shape
    V, C = embedding.shape
    H = C // 2
    H4, H8 = 4 * H, 8 * H
    cnn = [(cnn0_w_eff, cnn0_bias, cnn0_gamma, cnn0_beta),
           (cnn1_w_eff, cnn1_bias, cnn1_gamma, cnn1_beta),
           (cnn2_w_eff, cnn2_bias, cnn2_gamma, cnn2_beta)]
    depth = len(cnn)
    ksize = cnn0_w_eff.shape[-1]
    p = (ksize - 1) // 2

    emb_bf16 = embedding.astype(jnp.bfloat16)
    wc = jnp.transpose(jnp.stack([w for w, _, _, _ in cnn]),
                       (0, 3, 2, 1)).astype(jnp.bfloat16)
    wc = wc.reshape(depth, ksize * C, C)
    bc = jnp.stack([b.reshape(1, C) for _, b, _, _ in cnn])
    g = jnp.stack([ga.reshape(1, C) for _, _, ga, _ in cnn])
    be = jnp.stack([bb.reshape(1, C) for _, _, _, bb in cnn])

    wi = jnp.transpose(jnp.stack([lstm_w_ih_f, lstm_w_ih_b]),
                       (2, 0, 1)).reshape(C, H8).astype(jnp.bfloat16)
    bi = jnp.concatenate([lstm_b_ih_f + lstm_b_hh_f,
                          lstm_b_ih_b + lstm_b_hh_b]).reshape(1, H8)
    wh2 = jnp.transpose(jnp.stack([lstm_w_hh_f, lstm_w_hh_b]),
                        (0, 2, 1)).astype(jnp.bfloat16)

    tok3 = tokens.astype(jnp.int32).T.reshape(T, B, 1)
    lens3 = input_lengths.astype(jnp.int32).reshape(1, B, 1)

    Bblk = 16 if B % 16 == 0 else B
    nb = B // Bblk
    fe = functools.partial(_frontend_kernel, depth=depth, ksize=ksize,
                           eps=1e-5, neg_slope=0.2)
    z = pl.pallas_call(
        fe,
        grid=(nb,),
        in_specs=[
            pl.BlockSpec((1, Bblk, 1), lambda b: (0, b, 0)),
            pl.BlockSpec((T, Bblk, 1), lambda b: (0, b, 0)),
            _const_spec((V, C), lambda b: (0, 0)),
            _const_spec(wc.shape, lambda b: (0, 0, 0)),
            _const_spec(bc.shape, lambda b: (0, 0, 0)),
            _const_spec(g.shape, lambda b: (0, 0, 0)),
            _const_spec(be.shape, lambda b: (0, 0, 0)),
            _const_spec(wi.shape, lambda b: (0, 0)),
            _const_spec(bi.shape, lambda b: (0, 0)),
        ],
        out_specs=pl.BlockSpec((T, Bblk, H8), lambda b: (0, b, 0)),
        out_shape=jax.ShapeDtypeStruct((T, B, H8), jnp.bfloat16),
        scratch_shapes=[pltpu.VMEM((T + 2 * p, Bblk, C), jnp.bfloat16)],
        compiler_params=pltpu.CompilerParams(
            dimension_semantics=("arbitrary",),
            vmem_limit_bytes=_VMEM_LIMIT_BYTES),
    )(lens3, tok3, emb_bf16, wc, bc, g, be, wi, bi)

    nc = 4 if T % 8 == 0 else 1
    Tt = T // nc
    bl = functools.partial(_bilstm_kernel, t_total=T)
    of, ob = pl.pallas_call(
        bl,
        grid=(nc,),
        in_specs=[
            pl.BlockSpec((1, B, 1), lambda i: (0, 0, 0)),
            pl.BlockSpec((Tt, B, H4), lambda i: (i, 0, 0)),
            pl.BlockSpec((Tt, B, H4),
                         lambda i: (nc - 1 - i, 0, 1)),
            _const_spec((2, H, H4), lambda i: (0, 0, 0)),
        ],
        out_specs=[
            pl.BlockSpec((Tt, B, H), lambda i: (i, 0, 0)),
            pl.BlockSpec((Tt, B, H), lambda i: (nc - 1 - i, 0, 0)),
        ],
        out_shape=(jax.ShapeDtypeStruct((T, B, H), jnp.bfloat16),
                   jax.ShapeDtypeStruct((T, B, H), jnp.bfloat16)),
        scratch_shapes=[pltpu.VMEM((B, H), jnp.bfloat16),
                        pltpu.VMEM((B, H), jnp.float32),
                        pltpu.VMEM((B, H), jnp.bfloat16),
                        pltpu.VMEM((B, H), jnp.float32)],
        compiler_params=pltpu.CompilerParams(
            dimension_semantics=("arbitrary",),
            vmem_limit_bytes=_VMEM_LIMIT_BYTES),
    )(lens3, z, z, wh2)

    Bb3 = 16 if B % 16 == 0 else B
    nb3 = B // Bb3
    return pl.pallas_call(
        _epilogue_kernel,
        grid=(nb3,),
        in_specs=[
            pl.BlockSpec((T, Bb3, H), lambda b: (0, b, 0)),
            pl.BlockSpec((T, Bb3, H), lambda b: (0, b, 0)),
        ],
        out_specs=pl.BlockSpec((Bb3, 2 * H, T), lambda b: (b, 0, 0)),
        out_shape=jax.ShapeDtypeStruct((B, 2 * H, T), jnp.float32),
        compiler_params=pltpu.CompilerParams(
            dimension_semantics=("parallel",),
            vmem_limit_bytes=_VMEM_LIMIT_BYTES),
    )(of, ob)
